# Initial kernel scaffold; baseline (speedup 1.0000x reference)
import jax
import jax.numpy as jnp
from jax.experimental import pallas as pl


def kernel(neighbor_index, vertices, feature_map, weights, bias, displacement):
    raise NotImplementedError("write your pallas kernel here")


# speedup vs baseline: 26.3976x; 26.3976x over previous
import functools

import jax
import jax.numpy as jnp
from jax import lax
from jax.experimental import pallas as pl
from jax.experimental.pallas import tpu as pltpu
from jax.experimental.pallas import tpu_sc as plsc


def _tc_project(fm2, weights, bias2, C):
    M, K = fm2.shape
    W2 = weights.shape[1]
    BM = 256
    assert M % BM == 0
    grid = (M // BM,)

    def body(x_ref, w_ref, b_ref, cen_ref, sup_ref):
        y = jnp.dot(x_ref[...], w_ref[...], preferred_element_type=jnp.float32)
        y = y + b_ref[...]
        cen_ref[...] = y[:, :C]
        sup_ref[...] = y[:, C:]

    return pl.pallas_call(
        body,
        grid=grid,
        in_specs=[
            pl.BlockSpec((BM, K), lambda i: (i, 0)),
            pl.BlockSpec((K, W2), lambda i: (0, 0)),
            pl.BlockSpec((1, W2), lambda i: (0, 0)),
        ],
        out_specs=[
            pl.BlockSpec((BM, C), lambda i: (i, 0)),
            pl.BlockSpec((BM, C), lambda i: (i, 0)),
        ],
        out_shape=[
            jax.ShapeDtypeStruct((M, C), jnp.float32),
            jax.ShapeDtypeStruct((M, C), jnp.float32),
        ],
    )(fm2, weights, bias2)


def _sc_fuse(idx_flat, vert_flat, sup, cen, disp):
    M, C = sup.shape
    NB = idx_flat.shape[0] // M
    L = 16
    NCH = C // L
    NC, NS = 2, 16
    NW = NC * NS
    G = 8
    assert M % (NW * G) == 0
    per_w = M // NW
    nblk = per_w // G

    mesh = plsc.VectorSubcoreMesh(core_axis_name="c", subcore_axis_name="s")

    assert nblk % 2 == 1

    @functools.partial(
        pl.kernel,
        mesh=mesh,
        compiler_params=pltpu.CompilerParams(needs_layout_passes=False),
        out_type=jax.ShapeDtypeStruct((M, C), jnp.float32),
        scratch_types=[
            pltpu.VMEM((2, G * NB), jnp.int32),
            pltpu.VMEM((G * NB, C), jnp.float32),
            pltpu.VMEM((G * NB, C), jnp.float32),
            pltpu.VMEM((2, G, C), jnp.float32),
            pltpu.VMEM((G, C), jnp.float32),
            pltpu.VMEM((3 * M + L,), jnp.float32),
            pltpu.VMEM((3, C), jnp.float32),
            pltpu.SemaphoreType.DMA,
            pltpu.SemaphoreType.DMA,
            pltpu.SemaphoreType.DMA,
            pltpu.SemaphoreType.DMA,
        ],
    )
    def body(idx_hbm, vert_hbm, sup_hbm, cen_hbm, disp_hbm, out_hbm,
             idx_v, rows_v0, rows_v1, cen_v, out_v, coords_v, disp_v,
             gsem0, gsem1, isem0, isem1):
        rows_v = (rows_v0, rows_v1)
        gsem = (gsem0, gsem1)
        isem = (isem0, isem1)
        cid = lax.axis_index("c")
        sid = lax.axis_index("s")
        wid = sid * NC + cid
        base = wid * per_w
        pltpu.sync_copy(vert_hbm, coords_v)
        pltpu.sync_copy(disp_hbm, disp_v)
        dreg = [[disp_v[d, pl.ds(c * L, L)] for d in range(3)] for c in range(NCH)]

        def fetch_idx_cen(blk, b):
            v0 = base + blk * G
            pltpu.async_copy(idx_hbm.at[pl.ds(v0 * NB, G * NB)], idx_v.at[b], isem[b])
            pltpu.async_copy(cen_hbm.at[pl.ds(v0, G)], cen_v.at[b], isem[b])

        def wait_idx_cen(b):
            pltpu.make_async_copy(idx_hbm.at[pl.ds(0, G * NB)], idx_v.at[b], isem[b]).wait()
            pltpu.make_async_copy(cen_hbm.at[pl.ds(0, G)], cen_v.at[b], isem[b]).wait()

        def start_gather(b):
            pltpu.async_copy(sup_hbm.at[idx_v.at[b]], rows_v[b], gsem[b])

        def wait_gather(b):
            pltpu.make_async_copy(sup_hbm.at[pl.ds(0, G * NB)], rows_v[b], gsem[b]).wait()

        def compute_store(blk, b):
            v0 = base + blk * G
            rows = rows_v[b]

            def vert(g, _):
                v = v0 + g
                xv = coords_v[pl.ds(v, L)][0]
                yv = coords_v[pl.ds(M + v, L)][0]
                zv = coords_v[pl.ds(2 * M + v, L)][0]
                i_vec = idx_v[b, pl.ds(g * NB, NB)]
                dxv = plsc.load_gather(coords_v, [i_vec]) - xv
                dyv = plsc.load_gather(coords_v, [i_vec + M]) - yv
                dzv = plsc.load_gather(coords_v, [i_vec + 2 * M]) - zv
                acc = [None] * NCH
                for n in range(NB):
                    dx = dxv[n]
                    dy = dyv[n]
                    dz = dzv[n]
                    for c in range(NCH):
                        d0, d1, d2 = dreg[c]
                        th = jnp.maximum(dx * d0 + dy * d1 + dz * d2, 0.0)
                        p = th * rows[g * NB + n, pl.ds(c * L, L)]
                        acc[c] = p if acc[c] is None else jnp.maximum(acc[c], p)
                for c in range(NCH):
                    out_v[g, pl.ds(c * L, L)] = cen_v[b, g, pl.ds(c * L, L)] + acc[c]
                return 0

            lax.fori_loop(0, G, vert, 0)
            pltpu.sync_copy(out_v, out_hbm.at[pl.ds(v0, G)])

        fetch_idx_cen(0, 0)
        wait_idx_cen(0)
        start_gather(0)
        fetch_idx_cen(1, 1)

        def pair(bb, _):
            for b in (0, 1):
                blk = 2 * bb + b
                wait_gather(b)
                wait_idx_cen(1 - b)
                start_gather(1 - b)
                compute_store(blk, b)
                nxt = jnp.minimum(blk + 2, nblk - 1)
                fetch_idx_cen(nxt, b)
            return 0

        lax.fori_loop(0, (nblk - 1) // 2, pair, 0)

        wait_gather(0)
        compute_store(nblk - 1, 0)
        wait_idx_cen(1)

    return body(idx_flat, vert_flat, sup, cen, disp)


def kernel(neighbor_index, vertices, feature_map, weights, bias, displacement):
    BS, V, NB = neighbor_index.shape
    IN_C = feature_map.shape[-1]
    W2 = weights.shape[1]
    DC = displacement.shape[1]
    C = W2 - DC
    assert DC == C
    M = BS * V

    NW, G = 32, 8
    Mp = ((M + NW * G - 1) // (NW * G)) * (NW * G)

    fm2 = feature_map.reshape(M, IN_C)
    fm2 = jnp.pad(fm2, ((0, Mp - M), (0, 0)))
    cen, sup = _tc_project(fm2, weights, bias.reshape(1, W2), C)
    offs = (jnp.arange(BS, dtype=jnp.int32) * V)[:, None, None]
    idx_flat = (neighbor_index + offs).reshape(M * NB)
    idx_flat = jnp.pad(idx_flat, (0, (Mp - M) * NB))
    vert_flat = vertices.reshape(M, 3).T.reshape(3, M)
    vert_flat = jnp.pad(vert_flat, ((0, 0), (0, Mp - M))).reshape(3 * Mp)
    vert_flat = jnp.concatenate([vert_flat, jnp.zeros((16,), jnp.float32)])
    out = _sc_fuse(idx_flat, vert_flat, sup, cen, displacement)
    return out[:M].reshape(BS, V, C)


# speedup vs baseline: 42.1943x; 1.5984x over previous
import functools

import jax
import jax.numpy as jnp
from jax import lax
from jax.experimental import pallas as pl
from jax.experimental.pallas import tpu as pltpu
from jax.experimental.pallas import tpu_sc as plsc


def _tc_project(fm2, weights, bias2, C):
    M, K = fm2.shape
    W2 = weights.shape[1]
    BM = 1000
    assert M % BM == 0
    grid = (M // BM,)

    def body(x_ref, w_ref, b_ref, cen_ref, sup_ref):
        y = jnp.dot(x_ref[...], w_ref[...], preferred_element_type=jnp.float32)
        y = y + b_ref[...]
        cen_ref[...] = y[:, :C]
        sup_ref[...] = y[:, C:]

    return pl.pallas_call(
        body,
        grid=grid,
        in_specs=[
            pl.BlockSpec((BM, K), lambda i: (i, 0)),
            pl.BlockSpec((K, W2), lambda i: (0, 0)),
            pl.BlockSpec((1, W2), lambda i: (0, 0)),
        ],
        out_specs=[
            pl.BlockSpec((BM, C), lambda i: (i, 0)),
            pl.BlockSpec((BM, C), lambda i: (i, 0)),
        ],
        out_shape=[
            jax.ShapeDtypeStruct((M, C), jnp.float32),
            jax.ShapeDtypeStruct((M, C), jnp.float32),
        ],
    )(fm2, weights, bias2)


def _sc_fuse(idx_flat, vert_flat, sup, cen, disp):
    M, C = sup.shape
    NB = idx_flat.shape[0] // M
    L = 16
    NCH = C // L
    NC, NS = 2, 16
    NW = NC * NS
    G = 8
    assert M % G == 0
    tot_blk = M // G
    nblk = -(-tot_blk // NW)

    mesh = plsc.VectorSubcoreMesh(core_axis_name="c", subcore_axis_name="s")

    assert nblk % 2 == 1

    @functools.partial(
        pl.kernel,
        mesh=mesh,
        compiler_params=pltpu.CompilerParams(needs_layout_passes=False),
        out_type=jax.ShapeDtypeStruct((M, C), jnp.float32),
        scratch_types=[
            pltpu.VMEM((2, G * NB), jnp.int32),
            pltpu.VMEM((G * NB, C), jnp.float32),
            pltpu.VMEM((G * NB, C), jnp.float32),
            pltpu.VMEM((2, G, C), jnp.float32),
            pltpu.VMEM((G, C), jnp.float32),
            pltpu.VMEM((3 * M + L,), jnp.float32),
            pltpu.VMEM((3, C), jnp.float32),
            pltpu.SemaphoreType.DMA,
            pltpu.SemaphoreType.DMA,
            pltpu.SemaphoreType.DMA,
            pltpu.SemaphoreType.DMA,
        ],
    )
    def body(idx_hbm, vert_hbm, sup_hbm, cen_hbm, disp_hbm, out_hbm,
             idx_v, rows_v0, rows_v1, cen_v, out_v, coords_v, disp_v,
             gsem0, gsem1, isem0, isem1):
        rows_v = (rows_v0, rows_v1)
        gsem = (gsem0, gsem1)
        isem = (isem0, isem1)
        cid = lax.axis_index("c")
        sid = lax.axis_index("s")
        wid = sid * NC + cid
        pltpu.sync_copy(vert_hbm, coords_v)
        pltpu.sync_copy(disp_hbm, disp_v)
        dreg = [[disp_v[d, pl.ds(c * L, L)] for d in range(3)] for c in range(NCH)]

        def v_origin(pos):
            return jnp.minimum(wid + NW * pos, tot_blk - 1) * G

        def fetch_idx_cen(blk, b):
            v0 = v_origin(blk)
            pltpu.async_copy(idx_hbm.at[pl.ds(v0 * NB, G * NB)], idx_v.at[b], isem[b])
            pltpu.async_copy(cen_hbm.at[pl.ds(v0, G)], cen_v.at[b], isem[b])

        def wait_idx_cen(b):
            pltpu.make_async_copy(idx_hbm.at[pl.ds(0, G * NB)], idx_v.at[b], isem[b]).wait()
            pltpu.make_async_copy(cen_hbm.at[pl.ds(0, G)], cen_v.at[b], isem[b]).wait()

        def start_gather(b):
            pltpu.async_copy(sup_hbm.at[idx_v.at[b]], rows_v[b], gsem[b])

        def wait_gather(b):
            pltpu.make_async_copy(sup_hbm.at[pl.ds(0, G * NB)], rows_v[b], gsem[b]).wait()

        def compute_store(blk, b):
            v0 = v_origin(blk)
            rows = rows_v[b]

            def vert(g, _):
                v = v0 + g
                xv = coords_v[pl.ds(v, L)][0]
                yv = coords_v[pl.ds(M + v, L)][0]
                zv = coords_v[pl.ds(2 * M + v, L)][0]
                i_vec = idx_v[b, pl.ds(g * NB, NB)]
                dxv = plsc.load_gather(coords_v, [i_vec]) - xv
                dyv = plsc.load_gather(coords_v, [i_vec + M]) - yv
                dzv = plsc.load_gather(coords_v, [i_vec + 2 * M]) - zv
                acc = [None] * NCH
                for n in range(NB):
                    dx = dxv[n]
                    dy = dyv[n]
                    dz = dzv[n]
                    for c in range(NCH):
                        d0, d1, d2 = dreg[c]
                        th = jnp.maximum(dx * d0 + dy * d1 + dz * d2, 0.0)
                        p = th * rows[g * NB + n, pl.ds(c * L, L)]
                        acc[c] = p if acc[c] is None else jnp.maximum(acc[c], p)
                for c in range(NCH):
                    out_v[g, pl.ds(c * L, L)] = cen_v[b, g, pl.ds(c * L, L)] + acc[c]
                return 0

            lax.fori_loop(0, G, vert, 0)
            pltpu.sync_copy(out_v, out_hbm.at[pl.ds(v0, G)])

        fetch_idx_cen(0, 0)
        wait_idx_cen(0)
        start_gather(0)
        fetch_idx_cen(1, 1)

        def pair(bb, _):
            for b in (0, 1):
                blk = 2 * bb + b
                wait_gather(b)
                wait_idx_cen(1 - b)
                start_gather(1 - b)
                compute_store(blk, b)
                fetch_idx_cen(jnp.minimum(blk + 2, nblk - 1), b)
            return 0

        lax.fori_loop(0, (nblk - 1) // 2, pair, 0)

        wait_gather(0)
        compute_store(nblk - 1, 0)
        wait_idx_cen(1)

    return body(idx_flat, vert_flat, sup, cen, disp)


def kernel(neighbor_index, vertices, feature_map, weights, bias, displacement):
    BS, V, NB = neighbor_index.shape
    IN_C = feature_map.shape[-1]
    W2 = weights.shape[1]
    DC = displacement.shape[1]
    C = W2 - DC
    assert DC == C
    M = BS * V

    fm2 = feature_map.reshape(M, IN_C)
    cen, sup = _tc_project(fm2, weights, bias.reshape(1, W2), C)
    offs = (jnp.arange(BS, dtype=jnp.int32) * V)[:, None, None]
    idx_flat = (neighbor_index + offs).reshape(M * NB)
    vert_flat = vertices.reshape(M, 3).T.reshape(3 * M)
    vert_flat = jnp.concatenate([vert_flat, jnp.zeros((16,), jnp.float32)])
    out = _sc_fuse(idx_flat, vert_flat, sup, cen, displacement)
    return out.reshape(BS, V, C)


# speedup vs baseline: 51.2354x; 1.2143x over previous
import functools

import jax
import jax.numpy as jnp
from jax import lax
from jax.experimental import pallas as pl
from jax.experimental.pallas import tpu as pltpu
from jax.experimental.pallas import tpu_sc as plsc


def _tc_project(fm2, weights, bias2, C):
    M, K = fm2.shape
    W2 = weights.shape[1]
    BM = 1000
    assert M % BM == 0
    grid = (M // BM,)

    def body(x_ref, w_ref, b_ref, cen_ref, sup_ref):
        y = jnp.dot(x_ref[...], w_ref[...], preferred_element_type=jnp.float32)
        y = y + b_ref[...]
        cen_ref[...] = y[:, :C]
        sup_ref[...] = y[:, C:]

    return pl.pallas_call(
        body,
        grid=grid,
        in_specs=[
            pl.BlockSpec((BM, K), lambda i: (i, 0)),
            pl.BlockSpec((K, W2), lambda i: (0, 0)),
            pl.BlockSpec((1, W2), lambda i: (0, 0)),
        ],
        out_specs=[
            pl.BlockSpec((BM, C), lambda i: (i, 0)),
            pl.BlockSpec((BM, C), lambda i: (i, 0)),
        ],
        out_shape=[
            jax.ShapeDtypeStruct((M, C), jnp.float32),
            jax.ShapeDtypeStruct((M, C), jnp.float32),
        ],
    )(fm2, weights, bias2)


def _sc_fuse(idx_flat, vert_flat, sup, cen, disp):
    M, C = cen.shape
    NB = idx_flat.shape[0] // M
    L = 16
    NCH = C // L
    NC, NS = 2, 16
    NW = NC * NS
    G = 8
    assert M % G == 0
    tot_blk = M // G
    nblk = -(-tot_blk // NW)

    mesh = plsc.VectorSubcoreMesh(core_axis_name="c", subcore_axis_name="s")

    assert nblk % 2 == 1

    @functools.partial(
        pl.kernel,
        mesh=mesh,
        compiler_params=pltpu.CompilerParams(needs_layout_passes=False),
        out_type=jax.ShapeDtypeStruct((M, C), jnp.float32),
        scratch_types=[
            pltpu.VMEM((2, G * NB), jnp.int32),
            pltpu.VMEM((G * NB, C), jnp.float32),
            pltpu.VMEM((G * NB, C), jnp.float32),
            pltpu.VMEM((2, G, C), jnp.float32),
            pltpu.VMEM((G, C), jnp.float32),
            pltpu.VMEM((3 * M + L,), jnp.float32),
            pltpu.VMEM((3, C), jnp.float32),
            pltpu.SemaphoreType.DMA,
            pltpu.SemaphoreType.DMA,
            pltpu.SemaphoreType.DMA,
            pltpu.SemaphoreType.DMA,
        ],
    )
    def body(idx_hbm, vert_hbm, sup_hbm, cen_hbm, disp_hbm, out_hbm,
             idx_v, rows_v0, rows_v1, cen_v, out_v, coords_v, disp_v,
             gsem0, gsem1, isem0, isem1):
        rows_v = (rows_v0, rows_v1)
        gsem = (gsem0, gsem1)
        isem = (isem0, isem1)
        cid = lax.axis_index("c")
        sid = lax.axis_index("s")
        wid = sid * NC + cid
        pltpu.sync_copy(vert_hbm, coords_v)
        pltpu.sync_copy(disp_hbm, disp_v)
        dpk = [[plsc.pack(disp_v[d, pl.ds(2 * q * L, L)],
                          disp_v[d, pl.ds((2 * q + 1) * L, L)],
                          format=plsc.PackFormat.INTERLEAVED)
                for d in range(3)] for q in range(NCH // 2)]

        def v_origin(pos):
            return jnp.minimum(wid + NW * pos, tot_blk - 1) * G

        def fetch_idx_cen(blk, b):
            v0 = v_origin(blk)
            pltpu.async_copy(idx_hbm.at[pl.ds(v0 * NB, G * NB)], idx_v.at[b], isem[b])
            pltpu.async_copy(cen_hbm.at[pl.ds(v0, G)], cen_v.at[b], isem[b])

        def wait_idx_cen(b):
            pltpu.make_async_copy(idx_hbm.at[pl.ds(0, G * NB)], idx_v.at[b], isem[b]).wait()
            pltpu.make_async_copy(cen_hbm.at[pl.ds(0, G)], cen_v.at[b], isem[b]).wait()

        def start_gather(b):
            pltpu.async_copy(sup_hbm.at[idx_v.at[b]], rows_v[b], gsem[b])

        def wait_gather(b):
            pltpu.make_async_copy(sup_hbm.at[pl.ds(0, G * NB)], rows_v[b], gsem[b]).wait()

        def compute_store(blk, b):
            v0 = v_origin(blk)
            rows = rows_v[b]

            def vert(g, _):
                v = v0 + g
                cv = coords_v[pl.ds(3 * v, L)]
                xv = cv[0]
                yv = cv[1]
                zv = cv[2]
                i_vec = idx_v[b, pl.ds(g * NB, NB)]
                i3 = i_vec * 3
                dxv = plsc.load_gather(coords_v, [i3]) - xv
                dyv = plsc.load_gather(coords_v, [i3 + 1]) - yv
                dzv = plsc.load_gather(coords_v, [i3 + 2]) - zv
                pdx = plsc.bitcast(
                    plsc.pack(dxv, dxv, format=plsc.PackFormat.INTERLEAVED), jnp.int32)
                pdy = plsc.bitcast(
                    plsc.pack(dyv, dyv, format=plsc.PackFormat.INTERLEAVED), jnp.int32)
                pdz = plsc.bitcast(
                    plsc.pack(dzv, dzv, format=plsc.PackFormat.INTERLEAVED), jnp.int32)
                acc = [None] * (NCH // 2)
                for n in range(NB):
                    dx = plsc.bitcast(
                        jnp.broadcast_to(pdx[n], (L,)), jnp.bfloat16)
                    dy = plsc.bitcast(
                        jnp.broadcast_to(pdy[n], (L,)), jnp.bfloat16)
                    dz = plsc.bitcast(
                        jnp.broadcast_to(pdz[n], (L,)), jnp.bfloat16)
                    for q in range(NCH // 2):
                        d0, d1, d2 = dpk[q]
                        th = jnp.maximum(dx * d0 + dy * d1 + dz * d2, 0.0)
                        spk = plsc.pack(
                            rows[g * NB + n, pl.ds(2 * q * L, L)],
                            rows[g * NB + n, pl.ds((2 * q + 1) * L, L)],
                            format=plsc.PackFormat.INTERLEAVED)
                        p = th * spk
                        acc[q] = p if acc[q] is None else jnp.maximum(acc[q], p)
                for q in range(NCH // 2):
                    acc_a, acc_b = plsc.unpack(
                        acc[q], format=plsc.PackFormat.INTERLEAVED,
                        preferred_element_type=jnp.float32)
                    out_v[g, pl.ds(2 * q * L, L)] = (
                        cen_v[b, g, pl.ds(2 * q * L, L)] + acc_a)
                    out_v[g, pl.ds((2 * q + 1) * L, L)] = (
                        cen_v[b, g, pl.ds((2 * q + 1) * L, L)] + acc_b)
                return 0

            lax.fori_loop(0, G, vert, 0)
            pltpu.sync_copy(out_v, out_hbm.at[pl.ds(v0, G)])

        fetch_idx_cen(0, 0)
        wait_idx_cen(0)
        start_gather(0)
        fetch_idx_cen(1, 1)

        def pair(bb, _):
            for b in (0, 1):
                blk = 2 * bb + b
                wait_idx_cen(1 - b)
                start_gather(1 - b)
                wait_gather(b)
                compute_store(blk, b)
                fetch_idx_cen(jnp.minimum(blk + 2, nblk - 1), b)
            return 0

        lax.fori_loop(0, (nblk - 1) // 2, pair, 0)

        wait_gather(0)
        compute_store(nblk - 1, 0)
        wait_idx_cen(1)

    return body(idx_flat, vert_flat, sup, cen, disp)


def kernel(neighbor_index, vertices, feature_map, weights, bias, displacement):
    BS, V, NB = neighbor_index.shape
    IN_C = feature_map.shape[-1]
    W2 = weights.shape[1]
    DC = displacement.shape[1]
    C = W2 - DC
    assert DC == C
    M = BS * V

    fm2 = feature_map.reshape(M, IN_C)
    cen, sup = _tc_project(fm2, weights, bias.reshape(1, W2), C)
    offs = (jnp.arange(BS, dtype=jnp.int32) * V)[:, None, None]
    idx_flat = (neighbor_index + offs).reshape(M * NB)
    vert_flat = vertices.reshape(3 * M)
    vert_flat = jnp.concatenate([vert_flat, jnp.zeros((16,), jnp.float32)])
    out = _sc_fuse(idx_flat, vert_flat, sup, cen, displacement)
    return out.reshape(BS, V, C)
